# baseline (speedup 1.0000x reference)
import jax
import jax.numpy as jnp
from jax.experimental import pallas as pl
from jax.experimental.pallas import tpu as pltpu

B, H, S, D, U = 8, 16, 2048, 128, 32
SLAB = S * D
PCHUNK = U * D
HALF = B * H * SLAB


def _body(pos_ref, kc, vc, kn, vn, out, sem1, sem2):
    big_k = pltpu.make_async_copy(kc, out.at[pl.ds(0, HALF)], sem1)
    big_v = pltpu.make_async_copy(vc, out.at[pl.ds(HALF, HALF)], sem1)
    big_k.start()
    big_v.start()
    big_k.wait()
    big_v.wait()

    patches = []
    for bh in range(B * H):
        pos_b = pos_ref[bh // H]
        doff = bh * SLAB + pos_b * D
        patches.append(pltpu.make_async_copy(
            kn.at[pl.ds(bh * PCHUNK, PCHUNK)],
            out.at[pl.ds(doff, PCHUNK)], sem2))
        patches.append(pltpu.make_async_copy(
            vn.at[pl.ds(bh * PCHUNK, PCHUNK)],
            out.at[pl.ds(HALF + doff, PCHUNK)], sem2))
    for p in patches:
        p.start()
    for p in patches:
        p.wait()


def kernel(k_new, v_new, cache_seqlens, qcache_seqlens, k_cache_buf, v_cache_buf):
    pos = (cache_seqlens - qcache_seqlens).astype(jnp.int32)
    out_flat = pl.pallas_call(
        _body,
        in_specs=[
            pl.BlockSpec(memory_space=pltpu.SMEM),
            pl.BlockSpec(memory_space=pl.ANY),
            pl.BlockSpec(memory_space=pl.ANY),
            pl.BlockSpec(memory_space=pl.ANY),
            pl.BlockSpec(memory_space=pl.ANY),
        ],
        out_specs=pl.BlockSpec(memory_space=pl.ANY),
        out_shape=jax.ShapeDtypeStruct((2 * HALF,), jnp.float32),
        scratch_shapes=[pltpu.SemaphoreType.DMA, pltpu.SemaphoreType.DMA],
    )(
        pos,
        k_cache_buf.reshape(-1),
        v_cache_buf.reshape(-1),
        k_new.reshape(-1),
        v_new.reshape(-1),
    )
    return out_flat.reshape(2, B, H, S, D)


# speedup vs baseline: 39.5867x; 39.5867x over previous
import jax
import jax.numpy as jnp
from jax import lax
from jax.experimental import pallas as pl
from jax.experimental.pallas import tpu as pltpu
from jax.experimental.pallas import tpu_sc as plsc

B, H, S, D, U = 8, 16, 2048, 128, 32
SLAB = S * D
PCHUNK = U * D
HALF = B * H * SLAB
CH = 16384
CHPS = SLAB // CH
NCK = 8 * CHPS


def _body(kc_hbm, vc_hbm, kn_hbm, vn_hbm, pos_hbm, out_hbm,
          pos_v, b0, b1, b2, b3, b4, pb,
          sg0, sg1, sg2, sg3, sg4, ss0, ss1, ss2, ss3, ss4, sp):
    c = lax.axis_index("c")
    s = lax.axis_index("s")
    wid = s * 2 + c
    b = wid // 4
    q = wid % 4

    pltpu.sync_copy(pos_hbm, pos_v)
    pos_b = pos_v[b][0]

    for j in range(4):
        noff = (b * H + q * 4 + j) * PCHUNK
        pltpu.async_copy(kn_hbm.at[pl.ds(noff, PCHUNK)],
                         pb.at[pl.ds((2 * j) * PCHUNK, PCHUNK)], sp)
        pltpu.async_copy(vn_hbm.at[pl.ds(noff, PCHUNK)],
                         pb.at[pl.ds((2 * j + 1) * PCHUNK, PCHUNK)], sp)

    bufs = (b0, b1, b2, b3, b4)
    sg = (sg0, sg1, sg2, sg3, sg4)
    ss = (ss0, ss1, ss2, ss3, ss4)
    srcs = ((kc_hbm, 0), (vc_hbm, HALF))
    NB = 5
    LEAD = 3

    def soff(local):
        return (b * H + q * 4 + local // CHPS) * SLAB + (local % CHPS) * CH

    def gather(src, local, k):
        pltpu.async_copy(src.at[pl.ds(soff(local), CH)], bufs[k], sg[k])

    def scatter(kvhalf, local, k):
        pltpu.async_copy(
            bufs[k], out_hbm.at[pl.ds(kvhalf + soff(local), CH)], ss[k])

    def wait_g(k):
        pltpu.make_async_copy(kc_hbm.at[pl.ds(0, CH)], bufs[k], sg[k]).wait()

    def wait_s(k):
        pltpu.make_async_copy(bufs[k], out_hbm.at[pl.ds(0, CH)], ss[k]).wait()

    def static_step(ch):
        kv, local = divmod(ch, NCK // 2)
        k = ch % NB
        wait_g(k)
        scatter(srcs[kv][1], local, k)
        gch = ch + LEAD
        if gch < NCK:
            kv2, local2 = divmod(gch, NCK // 2)
            if gch >= NB:
                wait_s(gch % NB)
            gather(srcs[kv2][0], local2, gch % NB)

    def make_body(kv):
        src, kvhalf = srcs[kv]

        def body(i, carry):
            for k in range(NB):
                local = NB * i + k - kv * (NCK // 2)
                wait_g(k)
                scatter(kvhalf, local, k)
                kn = (k + LEAD) % NB
                wait_s(kn)
                gather(src, local + LEAD, kn)
            return carry

        return body

    for ch in range(LEAD):
        kv2, local2 = divmod(ch, NCK // 2)
        gather(srcs[kv2][0], local2, ch % NB)
    for ch in range(NB):
        static_step(ch)
    lax.fori_loop(1, 12, make_body(0), None)
    for ch in range(60, 70):
        static_step(ch)
    lax.fori_loop(14, 25, make_body(1), None)
    for ch in range(NCK - 3, NCK):
        static_step(ch)
    for k in range(NB):
        wait_s(k)

    pltpu.make_async_copy(kn_hbm.at[pl.ds(0, 8 * PCHUNK)], pb, sp).wait()
    for j in range(4):
        doff = (b * H + q * 4 + j) * SLAB + pos_b * D
        pltpu.async_copy(pb.at[pl.ds((2 * j) * PCHUNK, PCHUNK)],
                         out_hbm.at[pl.ds(doff, PCHUNK)], sp)
        pltpu.async_copy(pb.at[pl.ds((2 * j + 1) * PCHUNK, PCHUNK)],
                         out_hbm.at[pl.ds(HALF + doff, PCHUNK)], sp)
    pltpu.make_async_copy(pb, kn_hbm.at[pl.ds(0, 8 * PCHUNK)], sp).wait()


def kernel(k_new, v_new, cache_seqlens, qcache_seqlens, k_cache_buf, v_cache_buf):
    pos = (cache_seqlens - qcache_seqlens).astype(jnp.int32)
    pos_by_batch = jnp.broadcast_to(pos[:, None], (B, 16))
    mesh = plsc.VectorSubcoreMesh(core_axis_name="c", subcore_axis_name="s")
    out_flat = pl.kernel(
        _body,
        out_type=jax.ShapeDtypeStruct((2 * HALF,), jnp.float32),
        mesh=mesh,
        scratch_types=[
            pltpu.VMEM((B, 16), jnp.int32),
            pltpu.VMEM((CH,), jnp.float32),
            pltpu.VMEM((CH,), jnp.float32),
            pltpu.VMEM((CH,), jnp.float32),
            pltpu.VMEM((CH,), jnp.float32),
            pltpu.VMEM((CH,), jnp.float32),
            pltpu.VMEM((8 * PCHUNK,), jnp.float32),
            pltpu.SemaphoreType.DMA,
            pltpu.SemaphoreType.DMA,
            pltpu.SemaphoreType.DMA,
            pltpu.SemaphoreType.DMA,
            pltpu.SemaphoreType.DMA,
            pltpu.SemaphoreType.DMA,
            pltpu.SemaphoreType.DMA,
            pltpu.SemaphoreType.DMA,
            pltpu.SemaphoreType.DMA,
            pltpu.SemaphoreType.DMA,
            pltpu.SemaphoreType.DMA,
        ],
    )(
        k_cache_buf.reshape(-1),
        v_cache_buf.reshape(-1),
        k_new.reshape(-1),
        v_new.reshape(-1),
        pos_by_batch,
    )
    return out_flat.reshape(2, B, H, S, D)


# speedup vs baseline: 39.9817x; 1.0100x over previous
import jax
import jax.numpy as jnp
from jax import lax
from jax.experimental import pallas as pl
from jax.experimental.pallas import tpu as pltpu
from jax.experimental.pallas import tpu_sc as plsc

B, H, S, D, U = 8, 16, 2048, 128, 32
SLAB = S * D
PCHUNK = U * D
HALF = B * H * SLAB
CH = 16384
CHPS = SLAB // CH
NCK = 8 * CHPS


def _body(kc_hbm, vc_hbm, kn_hbm, vn_hbm, pos_hbm, out_hbm,
          pos_v, b0, b1, b2, b3, b4, pb,
          sg0, sg1, sg2, sg3, sg4, ss0, ss1, ss2, ss3, ss4, sp):
    c = lax.axis_index("c")
    s = lax.axis_index("s")
    wid = s * 2 + c
    b = wid // 4
    q = wid % 4

    bufs = (b0, b1, b2, b3, b4)
    sg = (sg0, sg1, sg2, sg3, sg4)
    ss = (ss0, ss1, ss2, ss3, ss4)
    srcs = ((kc_hbm, 0), (vc_hbm, HALF))
    NB = 5
    LEAD = 3

    def soff(local):
        return (b * H + q * 4 + local // CHPS) * SLAB + (local % CHPS) * CH

    def gather(src, local, k):
        pltpu.async_copy(src.at[pl.ds(soff(local), CH)], bufs[k], sg[k])

    def scatter(kvhalf, local, k):
        pltpu.async_copy(
            bufs[k], out_hbm.at[pl.ds(kvhalf + soff(local), CH)], ss[k])

    def wait_g(k):
        pltpu.make_async_copy(kc_hbm.at[pl.ds(0, CH)], bufs[k], sg[k]).wait()

    def wait_s(k):
        pltpu.make_async_copy(bufs[k], out_hbm.at[pl.ds(0, CH)], ss[k]).wait()

    def static_step(ch):
        kv, local = divmod(ch, NCK // 2)
        k = ch % NB
        wait_g(k)
        scatter(srcs[kv][1], local, k)
        gch = ch + LEAD
        if gch < NCK:
            kv2, local2 = divmod(gch, NCK // 2)
            if gch >= NB:
                wait_s(gch % NB)
            gather(srcs[kv2][0], local2, gch % NB)

    def make_body(kv):
        src, kvhalf = srcs[kv]

        def body(i, carry):
            for k in range(NB):
                local = NB * i + k - kv * (NCK // 2)
                wait_g(k)
                scatter(kvhalf, local, k)
                kn = (k + LEAD) % NB
                wait_s(kn)
                gather(src, local + LEAD, kn)
            return carry

        return body

    for ch in range(LEAD):
        kv2, local2 = divmod(ch, NCK // 2)
        gather(srcs[kv2][0], local2, ch % NB)

    pltpu.sync_copy(pos_hbm, pos_v)
    pos_b = pos_v[b][0]
    for j in range(4):
        noff = (b * H + q * 4 + j) * PCHUNK
        pltpu.async_copy(kn_hbm.at[pl.ds(noff, PCHUNK)],
                         pb.at[pl.ds((2 * j) * PCHUNK, PCHUNK)], sp)
        pltpu.async_copy(vn_hbm.at[pl.ds(noff, PCHUNK)],
                         pb.at[pl.ds((2 * j + 1) * PCHUNK, PCHUNK)], sp)

    for ch in range(NB):
        static_step(ch)
    lax.fori_loop(1, 12, make_body(0), None)
    for ch in range(60, 70):
        static_step(ch)
    lax.fori_loop(14, 25, make_body(1), None)
    for ch in range(NCK - 3, NCK):
        static_step(ch)
    for k in range(NB):
        wait_s(k)

    pltpu.make_async_copy(kn_hbm.at[pl.ds(0, 8 * PCHUNK)], pb, sp).wait()
    for j in range(4):
        doff = (b * H + q * 4 + j) * SLAB + pos_b * D
        pltpu.async_copy(pb.at[pl.ds((2 * j) * PCHUNK, PCHUNK)],
                         out_hbm.at[pl.ds(doff, PCHUNK)], sp)
        pltpu.async_copy(pb.at[pl.ds((2 * j + 1) * PCHUNK, PCHUNK)],
                         out_hbm.at[pl.ds(HALF + doff, PCHUNK)], sp)
    pltpu.make_async_copy(pb, kn_hbm.at[pl.ds(0, 8 * PCHUNK)], sp).wait()


def kernel(k_new, v_new, cache_seqlens, qcache_seqlens, k_cache_buf, v_cache_buf):
    pos = (cache_seqlens - qcache_seqlens).astype(jnp.int32)
    pos_by_batch = jnp.broadcast_to(pos[:, None], (B, 16))
    mesh = plsc.VectorSubcoreMesh(core_axis_name="c", subcore_axis_name="s")
    out_flat = pl.kernel(
        _body,
        out_type=jax.ShapeDtypeStruct((2 * HALF,), jnp.float32),
        mesh=mesh,
        scratch_types=[
            pltpu.VMEM((B, 16), jnp.int32),
            pltpu.VMEM((CH,), jnp.float32),
            pltpu.VMEM((CH,), jnp.float32),
            pltpu.VMEM((CH,), jnp.float32),
            pltpu.VMEM((CH,), jnp.float32),
            pltpu.VMEM((CH,), jnp.float32),
            pltpu.VMEM((8 * PCHUNK,), jnp.float32),
            pltpu.SemaphoreType.DMA,
            pltpu.SemaphoreType.DMA,
            pltpu.SemaphoreType.DMA,
            pltpu.SemaphoreType.DMA,
            pltpu.SemaphoreType.DMA,
            pltpu.SemaphoreType.DMA,
            pltpu.SemaphoreType.DMA,
            pltpu.SemaphoreType.DMA,
            pltpu.SemaphoreType.DMA,
            pltpu.SemaphoreType.DMA,
            pltpu.SemaphoreType.DMA,
        ],
    )(
        k_cache_buf.reshape(-1),
        v_cache_buf.reshape(-1),
        k_new.reshape(-1),
        v_new.reshape(-1),
        pos_by_batch,
    )
    return out_flat.reshape(2, B, H, S, D)
